# baseline (speedup 1.0000x reference)
import functools

import jax
import jax.numpy as jnp
from jax import lax
from jax.experimental import pallas as pl
from jax.experimental.pallas import tpu as pltpu
from jax.experimental.pallas import tpu_sc as plsc

_CHUNK = 64


def kernel(x, weight):
    batch, seq = x.shape
    dim = weight.shape[1]
    info = plsc.get_sparse_core_info()
    nw = info.num_cores * info.num_subcores
    rows_per_w = seq // nw
    nchunk = rows_per_w // _CHUNK

    mesh = plsc.VectorSubcoreMesh(core_axis_name="c", subcore_axis_name="s")

    @functools.partial(
        pl.kernel,
        mesh=mesh,
        out_type=jax.ShapeDtypeStruct((batch, seq, dim), weight.dtype),
        scratch_types=[
            pltpu.VMEM((_CHUNK, dim), weight.dtype),
            pltpu.SemaphoreType.DMA,
        ],
    )
    def _sc_bcast(w_hbm, o_hbm, buf, sem):
        wid = lax.axis_index("s") * info.num_cores + lax.axis_index("c")
        base = wid * rows_per_w

        def body(i, carry):
            r0 = base + i * _CHUNK
            pltpu.sync_copy(w_hbm.at[pl.ds(r0, _CHUNK), :], buf)
            writes = [
                pltpu.make_async_copy(
                    buf, o_hbm.at[b, pl.ds(r0, _CHUNK), :], sem
                )
                for b in range(batch)
            ]
            for w in writes:
                w.start()
            for w in writes:
                w.wait()
            return carry

        lax.fori_loop(0, nchunk, body, 0)

    return _sc_bcast(weight)


# speedup vs baseline: 1.0162x; 1.0162x over previous
import functools

import jax
import jax.numpy as jnp
from jax import lax
from jax.experimental import pallas as pl
from jax.experimental.pallas import tpu as pltpu
from jax.experimental.pallas import tpu_sc as plsc

_BUF_ROWS = 96


def _chunks(total, size):
    offs, sizes, o = [], [], 0
    while o < total:
        c = min(size, total - o)
        offs.append(o)
        sizes.append(c)
        o += c
    return offs, sizes


def kernel(x, weight):
    batch, seq = x.shape
    dim = weight.shape[1]
    info = plsc.get_sparse_core_info()
    nw = info.num_cores * info.num_subcores
    rows_per_w = seq // nw
    offs, sizes = _chunks(rows_per_w, _BUF_ROWS)

    mesh = plsc.VectorSubcoreMesh(core_axis_name="c", subcore_axis_name="s")

    @functools.partial(
        pl.kernel,
        mesh=mesh,
        out_type=jax.ShapeDtypeStruct((batch, seq, dim), weight.dtype),
        scratch_types=[
            pltpu.VMEM((_BUF_ROWS, dim), weight.dtype),
            pltpu.SemaphoreType.DMA,
        ],
    )
    def _sc_bcast(w_hbm, o_hbm, buf, sem):
        wid = lax.axis_index("s") * info.num_cores + lax.axis_index("c")
        base = wid * rows_per_w

        for off, size in zip(offs, sizes):
            r0 = base + off
            pltpu.sync_copy(w_hbm.at[pl.ds(r0, size), :], buf.at[pl.ds(0, size)])
            for b in range(batch):
                pltpu.sync_copy(
                    buf.at[pl.ds(0, size)], o_hbm.at[b, pl.ds(r0, size), :]
                )

    return _sc_bcast(weight)
